# baseline (speedup 1.0000x reference)
import functools

import jax
import jax.numpy as jnp
from jax import lax
from jax.experimental import pallas as pl
from jax.experimental.pallas import tpu as pltpu
from jax.experimental.pallas import tpu_sc as plsc

N = 512
E = 32768
OUT = 16
NC = 1
NS = 16
NW = NC * NS
EPW = E // NW
CH = 128
NCH = EPW // CH
LANES = 16
PK = 10512


def _rep(vec, i):
    return jnp.full((LANES,), vec[i])


def _vcn_sc(col2, ea2, pack):
    mesh = plsc.VectorSubcoreMesh(
        core_axis_name="c", subcore_axis_name="s",
        num_cores=NC, num_subcores=NS)

    @functools.partial(
        pl.kernel,
        out_type=jax.ShapeDtypeStruct((OUT,), jnp.float32),
        mesh=mesh,
        compiler_params=pltpu.CompilerParams(needs_layout_passes=False),
        scratch_types=[
            pltpu.VMEM((NCH, CH), jnp.int32),
            pltpu.VMEM((NCH, CH), jnp.float32),
            pltpu.VMEM((N,), jnp.float32),
            pltpu.VMEM_SHARED((N,), jnp.float32),
            pltpu.VMEM((PK,), jnp.float32),
            pltpu.VMEM((N,), jnp.float32),
            pltpu.VMEM((N,), jnp.float32),
            pltpu.VMEM((LANES,), jnp.float32),
            pltpu.SemaphoreType.DMA,
            pltpu.SemaphoreType.DMA,
        ],
    )
    def vcn(col_hbm, ea_hbm, pk_hbm, out_hbm, idx_v, val_v, zero_v, acc_sh,
            pk_v, s_v, u_v, outb_v, in_sem, sc_sem):
        s = lax.axis_index("s")
        cp_idx = pltpu.async_copy(col_hbm.at[s], idx_v, in_sem)
        cp_val = pltpu.async_copy(ea_hbm.at[s], val_v, in_sem)

        @pl.when(s == 0)
        def _():
            cp_pk = pltpu.async_copy(pk_hbm, pk_v, in_sem)
            for i in range(N // LANES):
                zero_v[pl.ds(i * LANES, LANES)] = jnp.zeros((LANES,), jnp.float32)
            pltpu.sync_copy(zero_v, acc_sh)
            cp_pk.wait()

        cp_idx.wait()
        cp_val.wait()
        plsc.subcore_barrier()

        cps = [
            pltpu.async_copy(val_v.at[j], acc_sh.at[idx_v.at[j]], sc_sem,
                             add=True)
            for j in range(NCH)
        ]
        for cp in cps:
            cp.wait()

        plsc.subcore_barrier()

        @pl.when(s == 0)
        def _():
            pltpu.sync_copy(acc_sh, s_v)

            slot = lambda t: pk_v[pl.ds(LANES * t, LANES)]
            relu = lambda v: jnp.maximum(v, 0.0)
            s016 = s_v[pl.ds(0, LANES)]

            p0 = slot(0) * slot(3)
            p1 = slot(1) * slot(4)
            p2 = slot(2) * slot(5)
            d = p0 * p2
            u_v[pl.ds(0, LANES)] = relu(p1 / d + d * s016 + slot(12))

            for wslot, bslot in ((6, 13), (9, 14)):
                uvec = u_v[pl.ds(0, LANES)]
                u0 = _rep(uvec, 0)
                u1 = _rep(uvec, 1)
                u2 = _rep(uvec, 2)
                q0 = u0 * slot(wslot)
                q1 = u1 * slot(wslot + 1)
                q2 = u2 * slot(wslot + 2)
                d = q0 * q2
                u_v[pl.ds(0, LANES)] = relu(q1 / d + d * s016 + slot(bslot))

            uvec = u_v[pl.ds(0, LANES)]
            u0 = _rep(uvec, 0)
            u1 = _rep(uvec, 1)
            u2 = _rep(uvec, 2)

            acc = pk_v[pl.ds(10496, LANES)]
            for j in range(N // LANES):
                o = j * LANES
                q0 = pk_v[pl.ds(256 + o, LANES)] * u0
                q1 = pk_v[pl.ds(768 + o, LANES)] * u1
                q2 = pk_v[pl.ds(1280 + o, LANES)] * u2
                d = q0 * q2
                h4 = q1 / d + d * s_v[pl.ds(o, LANES)] + pk_v[pl.ds(1792 + o, LANES)]
                for l in range(LANES):
                    acc = acc + _rep(h4, l) * pk_v[pl.ds(2304 + (o + l) * LANES, LANES)]
            outb_v[...] = acc
            pltpu.sync_copy(outb_v, out_hbm)

    return vcn(col2, ea2, pack)


def _pad3(v):
    return jnp.concatenate([v, jnp.zeros((13,), jnp.float32)])


def kernel(x, edge_index, batch, edge_attr, W0, b0, W1, b1, W2, b2, W3, b3,
           Wlin, blin):
    col2 = edge_index[1].reshape(NW, NCH, CH)
    ea2 = edge_attr.reshape(NW, NCH, CH)
    pack = jnp.concatenate([
        _pad3(x[0:3, 0]), _pad3(x[0:3, 1]), _pad3(x[0:3, 2]),
        _pad3(W0[0:3, 0]), _pad3(W0[0:3, 1]), _pad3(W0[0:3, 2]),
        _pad3(W1[0:3, 0]), _pad3(W1[0:3, 1]), _pad3(W1[0:3, 2]),
        _pad3(W2[0:3, 0]), _pad3(W2[0:3, 1]), _pad3(W2[0:3, 2]),
        _pad3(b0[0:3]), _pad3(b1[0:3]), _pad3(b2[0:3]),
        jnp.zeros((16,), jnp.float32),
        W3[:, 0], W3[:, 1], W3[:, 2], b3,
        Wlin.T.reshape(-1),
        blin,
    ])
    return _vcn_sc(col2, ea2, pack)
